# baseline (speedup 1.0000x reference)
import functools

import jax
import jax.numpy as jnp
from jax import lax
from jax.experimental import pallas as pl
from jax.experimental.pallas import tpu as pltpu
from jax.experimental.pallas import tpu_sc as plsc

VOCAB = 100000
EMBED = 128
BATCH = 4096
HIST = 50

NC = 2
NS = 16
NW = NC * NS

B_TOTAL = BATCH * HIST
B_PER_W = B_TOTAL // NW
CHUNK = 256
NBUF = 2
N_CHUNKS = B_PER_W // CHUNK

_mesh = plsc.VectorSubcoreMesh(core_axis_name="c", subcore_axis_name="s")


@functools.partial(
    pl.kernel,
    out_type=jax.ShapeDtypeStruct((B_TOTAL, EMBED), jnp.float32),
    mesh=_mesh,
    compiler_params=pltpu.CompilerParams(use_tc_tiling_on_sc=True),
    scratch_types=(
        [pltpu.VMEM((B_PER_W,), jnp.int32),
         pltpu.VMEM((NBUF, CHUNK, EMBED), jnp.float32),
         pltpu.VMEM_SHARED((NS, CHUNK, EMBED), jnp.float32)]
        + [pltpu.SemaphoreType.DMA] * (3 * NBUF)
    ),
)
def _gather_kernel(idx_hbm, table_hbm, out_hbm, idx_v, rows_v, shared, *sems):
    sid = lax.axis_index("s")
    wid = sid * NC + lax.axis_index("c")
    base = wid * B_PER_W
    sems_g = sems[:NBUF]
    sems_o = sems[NBUF:2 * NBUF]
    sems_x = sems[2 * NBUF:]
    pltpu.sync_copy(idx_hbm.at[pl.ds(base, B_PER_W)], idx_v)

    def gather(c):
        b = c % NBUF
        return pltpu.async_copy(
            table_hbm.at[idx_v.at[pl.ds(c * CHUNK, CHUNK)]],
            rows_v.at[b], sems_g[b])

    def put(c):
        b = c % NBUF
        pltpu.async_copy(rows_v.at[b], shared.at[sid], sems_x[b]).wait()
        return pltpu.async_copy(
            shared.at[sid], out_hbm.at[pl.ds(base + c * CHUNK, CHUNK)],
            sems_o[b])

    g = [None] * N_CHUNKS
    o = [None] * N_CHUNKS
    for c in range(N_CHUNKS):
        if c >= NBUF:
            o[c - NBUF].wait()
        g[c] = gather(c)
        if c >= 1:
            g[c - 1].wait()
            o[c - 1] = put(c - 1)
    g[N_CHUNKS - 1].wait()
    o[N_CHUNKS - 1] = put(N_CHUNKS - 1)
    for c in range(max(0, N_CHUNKS - NBUF), N_CHUNKS):
        o[c].wait()


def kernel(indices, table):
    flat_t = jnp.transpose(indices).reshape(-1)
    out = _gather_kernel(flat_t, table)
    return jnp.transpose(out.reshape(HIST, BATCH, EMBED), (1, 0, 2))


# speedup vs baseline: 1.0506x; 1.0506x over previous
import functools

import jax
import jax.numpy as jnp
from jax import lax
from jax.experimental import pallas as pl
from jax.experimental.pallas import tpu as pltpu
from jax.experimental.pallas import tpu_sc as plsc

VOCAB = 100000
EMBED = 128
BATCH = 4096
HIST = 50

NC = 2
NS = 16
NW = NC * NS

B_TOTAL = BATCH * HIST
B_PER_W = B_TOTAL // NW
CHUNK = 400
NBUF = 2
N_CHUNKS = B_PER_W // CHUNK

_mesh = plsc.VectorSubcoreMesh(core_axis_name="c", subcore_axis_name="s")


@functools.partial(
    pl.kernel,
    out_type=jax.ShapeDtypeStruct((B_TOTAL, EMBED), jnp.float32),
    mesh=_mesh,
    compiler_params=pltpu.CompilerParams(use_tc_tiling_on_sc=True),
    scratch_types=[
        pltpu.VMEM((B_PER_W,), jnp.int32),
        pltpu.VMEM((NBUF * CHUNK, EMBED), jnp.float32),
        pltpu.SemaphoreType.DMA,
        pltpu.SemaphoreType.DMA,
    ],
)
def _gather_kernel(idx_hbm, table_hbm, out_hbm, idx_v, rows_v, sem_g, sem_o):
    wid = lax.axis_index("s") * NC + lax.axis_index("c")
    base = wid * B_PER_W
    pltpu.sync_copy(idx_hbm.at[pl.ds(base, B_PER_W)], idx_v)

    def gather(c):
        buf = lax.rem(c, NBUF) * CHUNK
        pltpu.async_copy(
            table_hbm.at[idx_v.at[pl.ds(c * CHUNK, CHUNK)]],
            rows_v.at[pl.ds(buf, CHUNK)], sem_g)

    def put(c):
        buf = lax.rem(c, NBUF) * CHUNK
        pltpu.async_copy(
            rows_v.at[pl.ds(buf, CHUNK)],
            out_hbm.at[pl.ds(base + c * CHUNK, CHUNK)], sem_o)

    def drain(sem):
        pltpu.make_async_copy(
            table_hbm.at[pl.ds(0, CHUNK)],
            rows_v.at[pl.ds(0, CHUNK)], sem).wait()

    gather(0)

    def body(c, carry):
        @pl.when(c >= 1)
        def _():
            drain(sem_o)
        @pl.when(c + 1 < N_CHUNKS)
        def _():
            gather(c + 1)
        drain(sem_g)
        put(c)
        return carry

    lax.fori_loop(0, N_CHUNKS, body, None)
    drain(sem_o)


def kernel(indices, table):
    flat_t = jnp.transpose(indices).reshape(-1)
    out = _gather_kernel(flat_t, table)
    return jnp.transpose(out.reshape(HIST, BATCH, EMBED), (1, 0, 2))


# speedup vs baseline: 1.0507x; 1.0002x over previous
import functools

import jax
import jax.numpy as jnp
from jax import lax
from jax.experimental import pallas as pl
from jax.experimental.pallas import tpu as pltpu
from jax.experimental.pallas import tpu_sc as plsc

VOCAB = 100000
EMBED = 128
BATCH = 4096
HIST = 50

NC = 2
NS = 16
NW = NC * NS

B_TOTAL = BATCH * HIST
B_PER_W = B_TOTAL // NW
CHUNK = 200
NBUF = 4
N_CHUNKS = B_PER_W // CHUNK

_mesh = plsc.VectorSubcoreMesh(core_axis_name="c", subcore_axis_name="s")


@functools.partial(
    pl.kernel,
    out_type=jax.ShapeDtypeStruct((B_TOTAL, EMBED), jnp.float32),
    mesh=_mesh,
    compiler_params=pltpu.CompilerParams(use_tc_tiling_on_sc=True),
    scratch_types=[
        pltpu.VMEM((B_PER_W,), jnp.int32),
        pltpu.VMEM((NBUF * CHUNK, EMBED), jnp.float32),
        pltpu.SemaphoreType.DMA,
        pltpu.SemaphoreType.DMA,
    ],
)
def _gather_kernel(idx_hbm, table_hbm, out_hbm, idx_v, rows_v, sem_g, sem_o):
    wid = lax.axis_index("s") * NC + lax.axis_index("c")
    base = wid * B_PER_W
    pltpu.sync_copy(idx_hbm.at[pl.ds(base, B_PER_W)], idx_v)

    def gather(c):
        buf = lax.rem(c, NBUF) * CHUNK
        pltpu.async_copy(
            table_hbm.at[idx_v.at[pl.ds(c * CHUNK, CHUNK)]],
            rows_v.at[pl.ds(buf, CHUNK)], sem_g)

    def put(c):
        buf = lax.rem(c, NBUF) * CHUNK
        pltpu.async_copy(
            rows_v.at[pl.ds(buf, CHUNK)],
            out_hbm.at[pl.ds(base + c * CHUNK, CHUNK)], sem_o)

    def drain(sem):
        pltpu.make_async_copy(
            table_hbm.at[pl.ds(0, CHUNK)],
            rows_v.at[pl.ds(0, CHUNK)], sem).wait()

    for k in range(min(NBUF - 1, N_CHUNKS)):
        gather(k)

    def body(c, carry):
        @pl.when(c >= 1)
        def _():
            drain(sem_o)
        @pl.when(c + NBUF - 1 < N_CHUNKS)
        def _():
            gather(c + NBUF - 1)
        drain(sem_g)
        put(c)
        return carry

    lax.fori_loop(0, N_CHUNKS, body, None)
    drain(sem_o)


def kernel(indices, table):
    flat_t = jnp.transpose(indices).reshape(-1)
    out = _gather_kernel(flat_t, table)
    return jnp.transpose(out.reshape(HIST, BATCH, EMBED), (1, 0, 2))


# speedup vs baseline: 1.0524x; 1.0015x over previous
import functools

import jax
import jax.numpy as jnp
from jax import lax
from jax.experimental import pallas as pl
from jax.experimental.pallas import tpu as pltpu
from jax.experimental.pallas import tpu_sc as plsc

VOCAB = 100000
EMBED = 128
BATCH = 4096
HIST = 50

NC = 2
NS = 16
NW = NC * NS

B_TOTAL = BATCH * HIST
B_PER_W = B_TOTAL // NW
CHUNK = 400
NBUF = 2
N_CHUNKS = B_PER_W // CHUNK

_mesh = plsc.VectorSubcoreMesh(core_axis_name="c", subcore_axis_name="s")


@functools.partial(
    pl.kernel,
    out_type=jax.ShapeDtypeStruct((B_TOTAL, EMBED), jnp.float32),
    mesh=_mesh,
    compiler_params=pltpu.CompilerParams(use_tc_tiling_on_sc=True),
    scratch_types=[
        pltpu.VMEM((B_PER_W,), jnp.int32),
        pltpu.VMEM((NBUF * CHUNK, EMBED), jnp.float32),
        pltpu.SemaphoreType.DMA,
        pltpu.SemaphoreType.DMA,
    ],
)
def _gather_kernel(idx_hbm, table_hbm, out_hbm, idx_v, rows_v, sem_g, sem_o):
    wid = lax.axis_index("s") * NC + lax.axis_index("c")
    base = wid * B_PER_W
    pltpu.sync_copy(idx_hbm.at[pl.ds(base, B_PER_W)], idx_v)

    def gather(c):
        buf = lax.rem(c, NBUF) * CHUNK
        pltpu.async_copy(
            table_hbm.at[idx_v.at[pl.ds(c * CHUNK, CHUNK)]],
            rows_v.at[pl.ds(buf, CHUNK)], sem_g)

    def put(c):
        buf = lax.rem(c, NBUF) * CHUNK
        pltpu.async_copy(
            rows_v.at[pl.ds(buf, CHUNK)],
            out_hbm.at[pl.ds(base + c * CHUNK, CHUNK)], sem_o)

    def drain(sem):
        pltpu.make_async_copy(
            table_hbm.at[pl.ds(0, CHUNK)],
            rows_v.at[pl.ds(0, CHUNK)], sem).wait()

    for k in range(min(NBUF - 1, N_CHUNKS)):
        gather(k)

    def body(c, carry):
        @pl.when(c >= 1)
        def _():
            drain(sem_o)
        @pl.when(c + NBUF - 1 < N_CHUNKS)
        def _():
            gather(c + NBUF - 1)
        drain(sem_g)
        put(c)
        return carry

    lax.fori_loop(0, N_CHUNKS, body, None)
    drain(sem_o)


def kernel(indices, table):
    flat_t = jnp.transpose(indices).reshape(-1)
    out = _gather_kernel(flat_t, table)
    return jnp.transpose(out.reshape(HIST, BATCH, EMBED), (1, 0, 2))
